# baseline (speedup 1.0000x reference)
import jax
import jax.numpy as jnp
from jax.experimental import pallas as pl
from jax.experimental.pallas import tpu as pltpu

N_ETAS = 8
TN = 1024


def _fused_kernel(xq_ref, xs_ref, xt_ref, eta_ref,
                  wq_ref, ws_ref, brq_ref, brs_ref,
                  wbq_ref, wbs_ref, bbq_ref, bbs_ref,
                  out_ref, wall_ref):
    nz = 64

    @pl.when(pl.program_id(0) == 0)
    def _compose():
        wall_ref[:, 0:nz] = jnp.concatenate(
            [wbq_ref[...].T, bbq_ref[...].T], axis=0).astype(jnp.bfloat16)
        wall_ref[:, nz:2 * nz] = jnp.concatenate(
            [wbs_ref[...].T, bbs_ref[...].T], axis=0).astype(jnp.bfloat16)

        def expert_cols(wr, br, wb, bb, e):
            w = wr[e]
            sh = jnp.concatenate(
                [w[2 * nz:3 * nz], w[3 * nz:], w[nz:2 * nz]], axis=0)
            v = sh + jax.lax.dot_general(
                wb, w[:nz], (((0,), (0,)), ((), ())),
                preferred_element_type=jnp.float32)
            beff = jax.lax.dot_general(
                bb, w[:nz], (((0,), (0,)), ((), ())),
                preferred_element_type=jnp.float32) + br[e:e + 1]
            return jnp.concatenate([v, beff], axis=0)

        for e in range(N_ETAS):
            base = 2 * nz * (e + 1)
            wall_ref[:, base:base + nz] = expert_cols(
                wq_ref[...], brq_ref[...], wbq_ref[...], bbq_ref[...],
                e).astype(jnp.bfloat16)
            wall_ref[:, base + nz:base + 2 * nz] = expert_cols(
                ws_ref[...], brs_ref[...], wbs_ref[...], bbs_ref[...],
                e).astype(jnp.bfloat16)

    X = jnp.concatenate(
        [xq_ref[...].astype(jnp.bfloat16),
         xs_ref[...].astype(jnp.bfloat16),
         xt_ref[...].astype(jnp.bfloat16),
         jnp.ones((1, TN), jnp.bfloat16)], axis=0)
    Rall = jax.lax.dot_general(
        wall_ref[...], X, (((0,), (0,)), ((), ())),
        preferred_element_type=jnp.float32)
    P = Rall[:128]
    R = [Rall[128 * (e + 1):128 * (e + 2)] for e in range(N_ETAS)]
    eta = eta_ref[...]
    b0 = (eta & 1) == 1
    b1 = (eta & 2) == 2
    b2 = (eta & 4) == 4
    t0 = jnp.where(b0, R[1], R[0])
    t1 = jnp.where(b0, R[3], R[2])
    t2 = jnp.where(b0, R[5], R[4])
    t3 = jnp.where(b0, R[7], R[6])
    u0 = jnp.where(b1, t1, t0)
    u1 = jnp.where(b1, t3, t2)
    out_ref[...] = P + jnp.where(b2, u1, u0)


def kernel(x_QT, x_SLI, x_SST, eta, W_base_QT, b_base_QT, W_base_SLI,
           b_base_SLI, W_ratio_QT, b_ratio_QT, W_ratio_SLI, b_ratio_SLI):
    nz, h, w = x_QT.shape
    N = h * w
    E, FEAT, _ = W_ratio_QT.shape
    C = 2 * nz
    K = 3 * nz + 1
    M = (E + 1) * C
    xq = x_QT.reshape(nz, N)
    xs = x_SLI.reshape(nz, N)
    xt = x_SST.reshape(nz, N)
    T = N // TN
    eta2 = eta.reshape(1, N).astype(jnp.int32)
    bbq = b_base_QT[:, None]
    bbs = b_base_SLI[:, None]

    full = lambda shape: pl.BlockSpec(shape, lambda t: (0,) * len(shape))
    out = pl.pallas_call(
        _fused_kernel,
        grid=(T,),
        in_specs=[
            pl.BlockSpec((nz, TN), lambda t: (0, t)),
            pl.BlockSpec((nz, TN), lambda t: (0, t)),
            pl.BlockSpec((nz, TN), lambda t: (0, t)),
            pl.BlockSpec((1, TN), lambda t: (0, t)),
            full((E, FEAT, nz)), full((E, FEAT, nz)),
            full((E, nz)), full((E, nz)),
            full((nz, 3 * nz)), full((nz, 3 * nz)),
            full((nz, 1)), full((nz, 1)),
        ],
        out_specs=pl.BlockSpec((C, TN), lambda t: (0, t)),
        out_shape=jax.ShapeDtypeStruct((C, N), jnp.float32),
        scratch_shapes=[pltpu.VMEM((K, M), jnp.bfloat16)],
    )(xq, xs, xt, eta2, W_ratio_QT, W_ratio_SLI, b_ratio_QT, b_ratio_SLI,
      W_base_QT, W_base_SLI, bbq, bbs)

    return out.reshape(2, nz, h, w)


# speedup vs baseline: 1.1563x; 1.1563x over previous
import jax
import jax.numpy as jnp
from jax.experimental import pallas as pl
from jax.experimental.pallas import tpu as pltpu

N_ETAS = 8
TN = 2048


def _fused_kernel(xq_ref, xs_ref, xt_ref, eta_ref,
                  wq_ref, ws_ref, brq_ref, brs_ref,
                  wbq_ref, wbs_ref, bbq_ref, bbs_ref,
                  out_ref, wall_ref):
    nz = 64

    @pl.when(pl.program_id(0) == 0)
    def _compose():
        wall_ref[:, 0:nz] = jnp.concatenate(
            [wbq_ref[...].T, bbq_ref[...].T], axis=0).astype(jnp.bfloat16)
        wall_ref[:, nz:2 * nz] = jnp.concatenate(
            [wbs_ref[...].T, bbs_ref[...].T], axis=0).astype(jnp.bfloat16)

        def expert_cols(wr, br, wb, bb, e):
            w = wr[e]
            sh = jnp.concatenate(
                [w[2 * nz:3 * nz], w[3 * nz:], w[nz:2 * nz]], axis=0)
            v = sh + jax.lax.dot_general(
                wb, w[:nz], (((0,), (0,)), ((), ())),
                preferred_element_type=jnp.float32)
            beff = jax.lax.dot_general(
                bb, w[:nz], (((0,), (0,)), ((), ())),
                preferred_element_type=jnp.float32) + br[e:e + 1]
            return jnp.concatenate([v, beff], axis=0)

        for e in range(N_ETAS):
            base = 2 * nz * (e + 1)
            wall_ref[:, base:base + nz] = expert_cols(
                wq_ref[...], brq_ref[...], wbq_ref[...], bbq_ref[...],
                e).astype(jnp.bfloat16)
            wall_ref[:, base + nz:base + 2 * nz] = expert_cols(
                ws_ref[...], brs_ref[...], wbs_ref[...], bbs_ref[...],
                e).astype(jnp.bfloat16)

    X = jnp.concatenate(
        [xq_ref[...], xs_ref[...], xt_ref[...],
         jnp.ones((1, TN), jnp.bfloat16)], axis=0)
    Rall = jax.lax.dot_general(
        wall_ref[...], X, (((0,), (0,)), ((), ())),
        preferred_element_type=jnp.float32).astype(jnp.bfloat16)
    P = Rall[:128]
    R = [Rall[128 * (e + 1):128 * (e + 2)] for e in range(N_ETAS)]
    eta = eta_ref[...]
    b0 = (eta & 1) == 1
    b1 = (eta & 2) == 2
    b2 = (eta & 4) == 4
    t0 = jnp.where(b0, R[1], R[0])
    t1 = jnp.where(b0, R[3], R[2])
    t2 = jnp.where(b0, R[5], R[4])
    t3 = jnp.where(b0, R[7], R[6])
    u0 = jnp.where(b1, t1, t0)
    u1 = jnp.where(b1, t3, t2)
    out_ref[...] = P + jnp.where(b2, u1, u0)


def kernel(x_QT, x_SLI, x_SST, eta, W_base_QT, b_base_QT, W_base_SLI,
           b_base_SLI, W_ratio_QT, b_ratio_QT, W_ratio_SLI, b_ratio_SLI):
    nz, h, w = x_QT.shape
    N = h * w
    E, FEAT, _ = W_ratio_QT.shape
    C = 2 * nz
    K = 3 * nz + 1
    M = (E + 1) * C
    xq = x_QT.reshape(nz, N).astype(jnp.bfloat16)
    xs = x_SLI.reshape(nz, N).astype(jnp.bfloat16)
    xt = x_SST.reshape(nz, N).astype(jnp.bfloat16)
    T = N // TN
    eta2 = eta.reshape(1, N).astype(jnp.int32)
    bbq = b_base_QT[:, None]
    bbs = b_base_SLI[:, None]

    full = lambda shape: pl.BlockSpec(shape, lambda t: (0,) * len(shape))
    out = pl.pallas_call(
        _fused_kernel,
        grid=(T,),
        in_specs=[
            pl.BlockSpec((nz, TN), lambda t: (0, t)),
            pl.BlockSpec((nz, TN), lambda t: (0, t)),
            pl.BlockSpec((nz, TN), lambda t: (0, t)),
            pl.BlockSpec((1, TN), lambda t: (0, t)),
            full((E, FEAT, nz)), full((E, FEAT, nz)),
            full((E, nz)), full((E, nz)),
            full((nz, 3 * nz)), full((nz, 3 * nz)),
            full((nz, 1)), full((nz, 1)),
        ],
        out_specs=pl.BlockSpec((C, TN), lambda t: (0, t)),
        out_shape=jax.ShapeDtypeStruct((C, N), jnp.bfloat16),
        scratch_shapes=[pltpu.VMEM((K, M), jnp.bfloat16)],
    )(xq, xs, xt, eta2, W_ratio_QT, W_ratio_SLI, b_ratio_QT, b_ratio_SLI,
      W_base_QT, W_base_SLI, bbq, bbs)

    return out.astype(jnp.float32).reshape(2, nz, h, w)


# speedup vs baseline: 1.2289x; 1.0628x over previous
import jax
import jax.numpy as jnp
from jax.experimental import pallas as pl
from jax.experimental.pallas import tpu as pltpu

N_ETAS = 8
TN = 2048


def _fused_kernel(x_ref, eta_ref,
                  wq_ref, ws_ref, brq_ref, brs_ref,
                  wbq_ref, wbs_ref, bbq_ref, bbs_ref,
                  out_ref, wall_ref):
    nz = 64

    @pl.when(pl.program_id(0) == 0)
    def _compose():
        def expert_cols(wr, br, wb, bb, e):
            w = wr[e]
            sh = jnp.concatenate(
                [w[2 * nz:3 * nz], w[3 * nz:], w[nz:2 * nz]], axis=0)
            v = sh + wb.T + jax.lax.dot_general(
                wb, w[:nz], (((0,), (0,)), ((), ())),
                preferred_element_type=jnp.float32)
            beff = bb.T + jax.lax.dot_general(
                bb, w[:nz], (((0,), (0,)), ((), ())),
                preferred_element_type=jnp.float32) + br[e:e + 1]
            return jnp.concatenate([v, beff], axis=0)

        for e in range(N_ETAS):
            base = 2 * nz * e
            wall_ref[:, base:base + nz] = expert_cols(
                wq_ref[...], brq_ref[...], wbq_ref[...], bbq_ref[...],
                e).astype(jnp.bfloat16)
            wall_ref[:, base + nz:base + 2 * nz] = expert_cols(
                ws_ref[...], brs_ref[...], wbs_ref[...], bbs_ref[...],
                e).astype(jnp.bfloat16)

    X = jnp.concatenate(
        [x_ref[...], jnp.ones((1, TN), jnp.bfloat16)], axis=0)
    Rall = jax.lax.dot_general(
        wall_ref[...], X, (((0,), (0,)), ((), ())),
        preferred_element_type=jnp.float32).astype(jnp.bfloat16)
    R = [Rall[128 * e:128 * (e + 1)] for e in range(N_ETAS)]
    eta = eta_ref[...]
    b0 = (eta & 1) == 1
    b1 = (eta & 2) == 2
    b2 = (eta & 4) == 4
    t0 = jnp.where(b0, R[1], R[0])
    t1 = jnp.where(b0, R[3], R[2])
    t2 = jnp.where(b0, R[5], R[4])
    t3 = jnp.where(b0, R[7], R[6])
    u0 = jnp.where(b1, t1, t0)
    u1 = jnp.where(b1, t3, t2)
    out_ref[...] = jnp.where(b2, u1, u0)


def kernel(x_QT, x_SLI, x_SST, eta, W_base_QT, b_base_QT, W_base_SLI,
           b_base_SLI, W_ratio_QT, b_ratio_QT, W_ratio_SLI, b_ratio_SLI):
    nz, h, w = x_QT.shape
    N = h * w
    E, FEAT, _ = W_ratio_QT.shape
    C = 2 * nz
    K = 3 * nz + 1
    M = E * C
    x3 = jnp.concatenate(
        [x_QT.reshape(nz, N), x_SLI.reshape(nz, N), x_SST.reshape(nz, N)],
        axis=0).astype(jnp.bfloat16)
    T = N // TN
    eta2 = eta.reshape(1, N).astype(jnp.int32)
    bbq = b_base_QT[:, None]
    bbs = b_base_SLI[:, None]

    full = lambda shape: pl.BlockSpec(shape, lambda t: (0,) * len(shape))
    out = pl.pallas_call(
        _fused_kernel,
        grid=(T,),
        in_specs=[
            pl.BlockSpec((3 * nz, TN), lambda t: (0, t)),
            pl.BlockSpec((1, TN), lambda t: (0, t)),
            full((E, FEAT, nz)), full((E, FEAT, nz)),
            full((E, nz)), full((E, nz)),
            full((nz, 3 * nz)), full((nz, 3 * nz)),
            full((nz, 1)), full((nz, 1)),
        ],
        out_specs=pl.BlockSpec((C, TN), lambda t: (0, t)),
        out_shape=jax.ShapeDtypeStruct((C, N), jnp.bfloat16),
        scratch_shapes=[pltpu.VMEM((K, M), jnp.bfloat16)],
    )(x3, eta2, W_ratio_QT, W_ratio_SLI, b_ratio_QT, b_ratio_SLI,
      W_base_QT, W_base_SLI, bbq, bbs)

    return out.astype(jnp.float32).reshape(2, nz, h, w)


# speedup vs baseline: 1.2387x; 1.0080x over previous
import jax
import jax.numpy as jnp
from jax.experimental import pallas as pl
from jax.experimental.pallas import tpu as pltpu

N_ETAS = 8
TN = 4096


def _fused_kernel(x_ref, eta_ref,
                  wq_ref, ws_ref, brq_ref, brs_ref,
                  wbq_ref, wbs_ref, bbq_ref, bbs_ref,
                  out_ref, wall_ref):
    nz = 64

    @pl.when(pl.program_id(0) == 0)
    def _compose():
        def expert_cols(wr, br, wb, bb, e):
            w = wr[e]
            sh = jnp.concatenate(
                [w[2 * nz:3 * nz], w[3 * nz:], w[nz:2 * nz]], axis=0)
            v = sh + wb.T + jax.lax.dot_general(
                wb, w[:nz], (((0,), (0,)), ((), ())),
                preferred_element_type=jnp.float32)
            beff = bb.T + jax.lax.dot_general(
                bb, w[:nz], (((0,), (0,)), ((), ())),
                preferred_element_type=jnp.float32) + br[e:e + 1]
            return jnp.concatenate([v, beff], axis=0)

        for e in range(N_ETAS):
            base = 2 * nz * e
            wall_ref[:, base:base + nz] = expert_cols(
                wq_ref[...], brq_ref[...], wbq_ref[...], bbq_ref[...],
                e).astype(jnp.bfloat16)
            wall_ref[:, base + nz:base + 2 * nz] = expert_cols(
                ws_ref[...], brs_ref[...], wbs_ref[...], bbs_ref[...],
                e).astype(jnp.bfloat16)

    X = jnp.concatenate(
        [x_ref[...], jnp.ones((1, TN), jnp.bfloat16)], axis=0)
    Rall = jax.lax.dot_general(
        wall_ref[...], X, (((0,), (0,)), ((), ())),
        preferred_element_type=jnp.float32).astype(jnp.bfloat16)
    R = [Rall[128 * e:128 * (e + 1)] for e in range(N_ETAS)]
    eta = eta_ref[...]
    b0 = (eta & 1) == 1
    b1 = (eta & 2) == 2
    b2 = (eta & 4) == 4
    t0 = jnp.where(b0, R[1], R[0])
    t1 = jnp.where(b0, R[3], R[2])
    t2 = jnp.where(b0, R[5], R[4])
    t3 = jnp.where(b0, R[7], R[6])
    u0 = jnp.where(b1, t1, t0)
    u1 = jnp.where(b1, t3, t2)
    out_ref[...] = jnp.where(b2, u1, u0)


def kernel(x_QT, x_SLI, x_SST, eta, W_base_QT, b_base_QT, W_base_SLI,
           b_base_SLI, W_ratio_QT, b_ratio_QT, W_ratio_SLI, b_ratio_SLI):
    nz, h, w = x_QT.shape
    N = h * w
    E, FEAT, _ = W_ratio_QT.shape
    C = 2 * nz
    K = 3 * nz + 1
    M = E * C
    x3 = jnp.concatenate(
        [x_QT.reshape(nz, N), x_SLI.reshape(nz, N), x_SST.reshape(nz, N)],
        axis=0).astype(jnp.bfloat16)
    T = N // TN
    eta2 = eta.reshape(1, N).astype(jnp.int32)
    bbq = b_base_QT[:, None]
    bbs = b_base_SLI[:, None]

    full = lambda shape: pl.BlockSpec(shape, lambda t: (0,) * len(shape))
    out = pl.pallas_call(
        _fused_kernel,
        grid=(T,),
        in_specs=[
            pl.BlockSpec((3 * nz, TN), lambda t: (0, t)),
            pl.BlockSpec((1, TN), lambda t: (0, t)),
            full((E, FEAT, nz)), full((E, FEAT, nz)),
            full((E, nz)), full((E, nz)),
            full((nz, 3 * nz)), full((nz, 3 * nz)),
            full((nz, 1)), full((nz, 1)),
        ],
        out_specs=pl.BlockSpec((C, TN), lambda t: (0, t)),
        out_shape=jax.ShapeDtypeStruct((C, N), jnp.bfloat16),
        scratch_shapes=[pltpu.VMEM((K, M), jnp.bfloat16)],
    )(x3, eta2, W_ratio_QT, W_ratio_SLI, b_ratio_QT, b_ratio_SLI,
      W_base_QT, W_base_SLI, bbq, bbs)

    return out.astype(jnp.float32).reshape(2, nz, h, w)
